# baseline (speedup 1.0000x reference)
import functools

import jax
import jax.numpy as jnp
from jax import lax
from jax.experimental import pallas as pl
from jax.experimental.pallas import tpu as pltpu
from jax.experimental.pallas import tpu_sc as plsc

_NBINS = 15
_CHUNK = 250
_LANES = 16
_NW = 32


def _sc_body(nchunks, logits_ref, labels_ref,
             cnt_out, cnf_out, acc_out, nin_out, ncor_out,
             buf, labbuf, prow, cnt, cnf, acc, nin, ncor):
    wid = lax.axis_index("s") * 2 + lax.axis_index("c")
    iota = lax.iota(jnp.int32, _LANES)
    ones = jnp.ones((_LANES,), jnp.float32)
    zeros = jnp.zeros((_LANES,), jnp.float32)
    lane0 = iota == 0
    mall = iota < _LANES
    m6 = iota >= 12

    def _zero_row(r, carry):
        sl = pl.ds(r * _LANES, _LANES)
        cnt[sl] = zeros
        cnf[sl] = zeros
        acc[sl] = zeros
        return carry
    lax.fori_loop(0, _NBINS * 8, _zero_row, 0)
    for lq in range(8):
        sl = pl.ds(lq * _LANES, _LANES)
        nin[sl] = zeros
        ncor[sl] = zeros

    def _scat_add(ref, idx, x, mask):
        plsc.addupdate_scatter(ref, [idx], x, mask=mask)

    def _shuf(x, k):
        return x.at[iota ^ k].get(mode="promise_in_bounds")

    def _allreduce(x, op):
        for k in (8, 4, 2, 1):
            x = op(x, _shuf(x, k))
        return x

    def _do_row(r, carry):
        v = [buf[r, pl.ds(16 * j, _LANES)] for j in range(6)]
        v6 = buf[r, pl.ds(84, _LANES)]
        mm = v[0]
        for j in range(1, 6):
            mm = jnp.maximum(mm, v[j])
        mm = jnp.maximum(mm, v6)
        mvec = _allreduce(mm, jnp.maximum)
        e = [jnp.exp(vj - mvec) for vj in v]
        e6 = jnp.exp(v6 - mvec)
        ssum = e[0]
        for j in range(1, 6):
            ssum = ssum + e[j]
        ssum = ssum + jnp.where(m6, e6, zeros)
        svec = _allreduce(ssum, jnp.add)
        p = [ej / svec for ej in e]
        p6 = e6 / svec
        pmax = ones / svec
        tmin = jnp.full((_LANES,), 1000, jnp.int32)
        for j in range(6):
            tmin = jnp.minimum(tmin, jnp.where(p[j] == pmax, iota + 16 * j, 1000))
        tmin = jnp.minimum(tmin, jnp.where((p6 == pmax) & m6, iota + 84, 1000))
        choice = _allreduce(tmin, jnp.minimum)

        for j in range(6):
            prow[pl.ds(16 * j, _LANES)] = p[j]
        prow[pl.ds(84, _LANES)] = p6

        for j in range(6):
            u = jnp.minimum((p[j] * 15.0).astype(jnp.int32), 14)
            idx = (u << 7) + (iota + 16 * j)
            _scat_add(cnt, idx, ones, mall)
            _scat_add(cnf, idx, p[j], mall)
        u6 = jnp.minimum((p6 * 15.0).astype(jnp.int32), 14)
        idx6 = (u6 << 7) + (iota + 84)
        _scat_add(cnt, idx6, ones, m6)
        _scat_add(cnf, idx6, p6, m6)

        lv = labbuf[pl.ds(r, _LANES)]
        lab_vec = lv.at[jnp.zeros((_LANES,), jnp.int32)].get(
            mode="promise_in_bounds")
        conf_lab = plsc.load_gather(prow, [lab_vec])
        ulab = jnp.minimum((conf_lab * 15.0).astype(jnp.int32), 14)
        _scat_add(acc, (ulab << 7) + lab_vec, ones, lane0)
        _scat_add(nin, lab_vec, ones, lane0)
        eq_vec = jnp.where(choice == lab_vec, 1.0, 0.0).astype(jnp.float32)
        _scat_add(ncor, lab_vec, eq_vec, lane0)
        return carry

    def _chunk_iter(j, carry):
        c = wid + _NW * j
        @pl.when(c < nchunks)
        def _():
            pltpu.sync_copy(logits_ref.at[c], buf)
            pltpu.sync_copy(labels_ref.at[c], labbuf)
            lax.fori_loop(0, _CHUNK, _do_row, 0)
        return carry
    njw = (nchunks + _NW - 1) // _NW
    lax.fori_loop(0, njw, _chunk_iter, 0)

    pltpu.sync_copy(cnt, cnt_out.at[wid])
    pltpu.sync_copy(cnf, cnf_out.at[wid])
    pltpu.sync_copy(acc, acc_out.at[wid])
    pltpu.sync_copy(nin, nin_out.at[wid])
    pltpu.sync_copy(ncor, ncor_out.at[wid])


def _tc_final_body(nrows, cnt_ref, cnf_ref, acc_ref, nin_ref, ncor_ref,
                   sce_ref, cacc_ref):
    cnt = jnp.sum(cnt_ref[...], axis=0)
    cnf = jnp.sum(cnf_ref[...], axis=0)
    acc = jnp.sum(acc_ref[...], axis=0)
    nin = jnp.sum(nin_ref[...], axis=0, keepdims=True)
    ncor = jnp.sum(ncor_ref[...], axis=0, keepdims=True)
    prop = cnt / float(nrows)
    safe = jnp.maximum(cnt, 1.0)
    contrib = jnp.where(cnt > 0.0,
                        jnp.abs(cnf / safe - acc / safe) * prop, 0.0)
    sce_ref[...] = jnp.sum(contrib, axis=0, keepdims=True)
    cacc_ref[...] = ncor / nin


def kernel(logits, labels):
    N, C = logits.shape
    nchunks = N // _CHUNK
    logits3 = logits.reshape(nchunks, _CHUNK, C)
    labels2 = jnp.pad(labels.reshape(nchunks, _CHUNK), ((0, 0), (0, 22)))

    mesh = plsc.VectorSubcoreMesh(core_axis_name="c", subcore_axis_name="s")
    sck = functools.partial(
        pl.kernel,
        mesh=mesh,
        compiler_params=pltpu.CompilerParams(needs_layout_passes=False),
        out_type=[
            jax.ShapeDtypeStruct((_NW, _NBINS * 128), jnp.float32),
            jax.ShapeDtypeStruct((_NW, _NBINS * 128), jnp.float32),
            jax.ShapeDtypeStruct((_NW, _NBINS * 128), jnp.float32),
            jax.ShapeDtypeStruct((_NW, 128), jnp.float32),
            jax.ShapeDtypeStruct((_NW, 128), jnp.float32),
        ],
        scratch_types=[
            pltpu.VMEM((_CHUNK, 100), jnp.float32),
            pltpu.VMEM((_CHUNK + 22,), jnp.int32),
            pltpu.VMEM((112,), jnp.float32),
            pltpu.VMEM((_NBINS * 128,), jnp.float32),
            pltpu.VMEM((_NBINS * 128,), jnp.float32),
            pltpu.VMEM((_NBINS * 128,), jnp.float32),
            pltpu.VMEM((128,), jnp.float32),
            pltpu.VMEM((128,), jnp.float32),
        ],
    )(functools.partial(_sc_body, nchunks))
    cnt, cnf, acc, nin, ncor = sck(logits3, labels2)
    cnt = cnt.reshape(_NW, _NBINS, 128)
    cnf = cnf.reshape(_NW, _NBINS, 128)
    acc = acc.reshape(_NW, _NBINS, 128)

    out = pl.pallas_call(
        functools.partial(_tc_final_body, N),
        out_shape=[
            jax.ShapeDtypeStruct((1, 128), jnp.float32),
            jax.ShapeDtypeStruct((1, 128), jnp.float32),
        ],
    )(cnt, cnf, acc, nin, ncor)
    return (out[0][0, :C], out[1][0, :C])
